# Initial kernel scaffold; baseline (speedup 1.0000x reference)
import jax
import jax.numpy as jnp
from jax.experimental import pallas as pl


def kernel(x_bd):
    raise NotImplementedError("write your pallas kernel here")


# speedup vs baseline: 3.0396x; 3.0396x over previous
import jax
import jax.numpy as jnp
from jax.experimental import pallas as pl

_B = 16384
_D = 512
_OUT_LO = 120
_OUT_HI = 136
_BLK = 2048


def _copy_set_body(x_ref, o_ref):
    lane = jax.lax.broadcasted_iota(jnp.int32, (_BLK, _D), 1)
    hit = (lane == _OUT_LO) | (lane == _OUT_HI)
    o_ref[...] = jnp.where(hit, jnp.float32(2.0), x_ref[...])


def kernel(x_bd):
    x2 = x_bd.reshape(_B, _D)
    out = pl.pallas_call(
        _copy_set_body,
        grid=(_B // _BLK,),
        in_specs=[pl.BlockSpec((_BLK, _D), lambda i: (i, 0))],
        out_specs=pl.BlockSpec((_BLK, _D), lambda i: (i, 0)),
        out_shape=jax.ShapeDtypeStruct((_B, _D), jnp.float32),
    )(x2)
    return out.reshape(_B, 1, _D)
